# Initial kernel scaffold; baseline (speedup 1.0000x reference)
import jax
import jax.numpy as jnp
from jax.experimental import pallas as pl


def kernel(x, y, patch_centers):
    raise NotImplementedError("write your pallas kernel here")


# speedup vs baseline: 6.8037x; 6.8037x over previous
import functools

import jax
import jax.numpy as jnp
from jax.experimental import pallas as pl

_BM = 256


def _topk_weighted(s, yfull, k):
    bm, n = s.shape
    iota = jax.lax.broadcasted_iota(jnp.int32, (bm, n), 1)
    v0 = None
    for step in range(k):
        m = jnp.max(s, axis=1, keepdims=True)
        cand = jnp.where(s == m, iota, n)
        ji = jnp.min(cand, axis=1, keepdims=True)
        sel = iota == ji
        if step == 0:
            v0 = m
            u = sel.astype(jnp.float32)
            z = jnp.ones_like(m)
        else:
            e = jnp.exp(m - v0)
            u = u + e * sel.astype(jnp.float32)
            z = z + e
        if step < k - 1:
            s = jnp.where(sel, -jnp.inf, s)
    w = u * (1.0 / z)
    return jnp.dot(w, yfull, preferred_element_type=jnp.float32)


def _body(x_ref, xt_ref, y_ref, pc_ref, pct_ref, sim_ref, spat_ref,
          *, sim_k, spat_k):
    yfull = y_ref[...]

    s = jnp.dot(x_ref[...], xt_ref[...], preferred_element_type=jnp.float32)
    sim_ref[...] = _topk_weighted(s, yfull, sim_k)

    pcb = pc_ref[...]
    pct = pct_ref[...]
    c2 = jnp.sum(pct * pct, axis=0, keepdims=True)
    c2b = jnp.sum(pcb * pcb, axis=1, keepdims=True)
    nd = -(c2b + c2
           - 2.0 * jnp.dot(pcb, pct, preferred_element_type=jnp.float32))
    spat_ref[...] = _topk_weighted(nd, yfull, spat_k)


def kernel(x, y, patch_centers):
    n, d_feat = x.shape
    d_y = y.shape[1]
    xt = x.T
    pct = patch_centers.T

    body = functools.partial(_body, sim_k=5, spat_k=4)
    sim_out, spat_out = pl.pallas_call(
        body,
        grid=(n // _BM,),
        in_specs=[
            pl.BlockSpec((_BM, d_feat), lambda i: (i, 0)),
            pl.BlockSpec((d_feat, n), lambda i: (0, 0)),
            pl.BlockSpec((n, d_y), lambda i: (0, 0)),
            pl.BlockSpec((_BM, 2), lambda i: (i, 0)),
            pl.BlockSpec((2, n), lambda i: (0, 0)),
        ],
        out_specs=[
            pl.BlockSpec((_BM, d_y), lambda i: (i, 0)),
            pl.BlockSpec((_BM, d_y), lambda i: (i, 0)),
        ],
        out_shape=[
            jax.ShapeDtypeStruct((n, d_y), jnp.float32),
            jax.ShapeDtypeStruct((n, d_y), jnp.float32),
        ],
    )(x, xt, y, patch_centers, pct)
    return jnp.stack([sim_out, spat_out], axis=0)


# speedup vs baseline: 6.9359x; 1.0194x over previous
import functools

import jax
import jax.numpy as jnp
from jax.experimental import pallas as pl

_BM = 256


def _topk_weighted(s, yfull, k):
    bm, n = s.shape
    iota = jax.lax.broadcasted_iota(jnp.int32, (bm, n), 1)
    v0 = None
    for step in range(k):
        m = jnp.max(s, axis=1, keepdims=True)
        ji = jnp.argmax(s, axis=1)[:, None]
        sel = iota == ji
        if step == 0:
            v0 = m
            u = sel.astype(jnp.float32)
            z = jnp.ones_like(m)
        else:
            e = jnp.exp(m - v0)
            u = u + e * sel.astype(jnp.float32)
            z = z + e
        if step < k - 1:
            s = jnp.where(sel, -jnp.inf, s)
    w = u * (1.0 / z)
    return jnp.dot(w, yfull, preferred_element_type=jnp.float32)


def _body(x_ref, xt_ref, y_ref, pc_ref, pct_ref, sim_ref, spat_ref,
          *, sim_k, spat_k):
    yfull = y_ref[...]

    s = jnp.dot(x_ref[...], xt_ref[...], preferred_element_type=jnp.float32)
    sim_ref[...] = _topk_weighted(s, yfull, sim_k)

    pcb = pc_ref[...]
    pct = pct_ref[...]
    c2 = jnp.sum(pct * pct, axis=0, keepdims=True)
    c2b = jnp.sum(pcb * pcb, axis=1, keepdims=True)
    nd = -(c2b + c2
           - 2.0 * jnp.dot(pcb, pct, preferred_element_type=jnp.float32))
    spat_ref[...] = _topk_weighted(nd, yfull, spat_k)


def kernel(x, y, patch_centers):
    n, d_feat = x.shape
    d_y = y.shape[1]
    xt = x.T
    pct = patch_centers.T

    body = functools.partial(_body, sim_k=5, spat_k=4)
    sim_out, spat_out = pl.pallas_call(
        body,
        grid=(n // _BM,),
        in_specs=[
            pl.BlockSpec((_BM, d_feat), lambda i: (i, 0)),
            pl.BlockSpec((d_feat, n), lambda i: (0, 0)),
            pl.BlockSpec((n, d_y), lambda i: (0, 0)),
            pl.BlockSpec((_BM, 2), lambda i: (i, 0)),
            pl.BlockSpec((2, n), lambda i: (0, 0)),
        ],
        out_specs=[
            pl.BlockSpec((_BM, d_y), lambda i: (i, 0)),
            pl.BlockSpec((_BM, d_y), lambda i: (i, 0)),
        ],
        out_shape=[
            jax.ShapeDtypeStruct((n, d_y), jnp.float32),
            jax.ShapeDtypeStruct((n, d_y), jnp.float32),
        ],
    )(x, xt, y, patch_centers, pct)
    return jnp.stack([sim_out, spat_out], axis=0)
